# baseline (speedup 1.0000x reference)
import functools

import jax
import jax.numpy as jnp
from jax import lax
from jax.experimental import pallas as pl
from jax.experimental.pallas import tpu as pltpu
from jax.experimental.pallas import tpu_sc as plsc

_NC = 2
_NS = 16
_NW = _NC * _NS
_LANES = 16


def _sc_gather(uid, iid, gmf_u, gmf_i, mlp_u, mlp_i):
    B = uid.shape[0]
    D = gmf_u.shape[1]
    bpw = B // _NW
    C = 64
    nchunks = bpw // C
    nbuf = 2

    mesh = plsc.VectorSubcoreMesh(core_axis_name="c", subcore_axis_name="s")
    fdim = jax.ShapeDtypeStruct((B, D), jnp.float32)
    cbuf = pltpu.VMEM((C, D), jnp.float32)
    @functools.partial(
        pl.kernel,
        out_type=[fdim, fdim, fdim],
        mesh=mesh,
        scratch_types=[
            pltpu.VMEM((bpw,), jnp.int32),
            pltpu.VMEM((bpw,), jnp.int32),
            [[cbuf] * 4] * nbuf,
            [pltpu.SemaphoreType.DMA] * nbuf,
            [pltpu.SemaphoreType.DMA] * nbuf,
        ],
    )
    def k(uid_hbm, iid_hbm, gu_hbm, gi_hbm, mu_hbm, mi_hbm,
          prod_out, mu_out, mi_out,
          uidx, iidx, bufs, gsem, wsem):
        wid = lax.axis_index("s") * _NC + lax.axis_index("c")
        base = wid * bpw
        pltpu.sync_copy(uid_hbm.at[pl.ds(base, bpw)], uidx)
        pltpu.sync_copy(iid_hbm.at[pl.ds(base, bpw)], iidx)

        def gathers(ci, s):
            off = ci * C
            cu = uidx.at[pl.ds(off, C)]
            cv = iidx.at[pl.ds(off, C)]
            bgu, bgi, bmu, bmi = bufs[s]
            return [pltpu.make_async_copy(gu_hbm.at[cu], bgu, gsem[s]),
                    pltpu.make_async_copy(gi_hbm.at[cv], bgi, gsem[s]),
                    pltpu.make_async_copy(mu_hbm.at[cu], bmu, gsem[s]),
                    pltpu.make_async_copy(mi_hbm.at[cv], bmi, gsem[s])]

        def writes(ci, s):
            off = base + ci * C
            bgu, _, bmu, bmi = bufs[s]
            return [pltpu.make_async_copy(bgu, prod_out.at[pl.ds(off, C)], wsem[s]),
                    pltpu.make_async_copy(bmu, mu_out.at[pl.ds(off, C)], wsem[s]),
                    pltpu.make_async_copy(bmi, mi_out.at[pl.ds(off, C)], wsem[s])]

        def process(ci, s):
            for c in gathers(ci, s):
                c.wait()
            bgu, bgi, _, _ = bufs[s]

            def row(j, carry2):
                for t in range(D // _LANES):
                    sl = pl.ds(t * _LANES, _LANES)
                    bgu[j, sl] = bgu[j, sl] * bgi[j, sl]
                return carry2

            lax.fori_loop(0, C, row, 0)
            for c in writes(ci, s):
                c.start()

        for ci in range(nchunks):
            s = ci % nbuf
            if ci >= nbuf:
                for c in writes(ci - nbuf, s):
                    c.wait()
            for c in gathers(ci, s):
                c.start()
            if ci > 0:
                process(ci - 1, (ci - 1) % nbuf)
        process(nchunks - 1, (nchunks - 1) % nbuf)
        for ci in range(nchunks - nbuf, nchunks):
            for c in writes(ci, ci % nbuf):
                c.wait()

    return k(uid, iid, gmf_u, gmf_i, mlp_u, mlp_i)


def _tc_tail(prod, mu, mi, gmf_Wp, W1, b1, W2, b2, ncf_W, ncf_b):
    B, D = mu.shape
    R = 8192
    grid = B // R

    def body(prod_ref, mu_ref, mi_ref, gmfW_ref, W1_ref, b1_ref, W2_ref,
             b2_ref, ncfW_ref, ncfb_ref, out_ref):
        f32 = jnp.float32
        wg = ncfW_ref[:, :64]
        wm = ncfW_ref[:, 64:]
        g = jax.lax.dot_general(wg, gmfW_ref[...], (((1,), (0,)), ((), ())),
                                preferred_element_type=f32)
        m = jax.lax.dot_general(wm, W2_ref[...], (((1,), (0,)), ((), ())),
                                preferred_element_type=f32)
        c = jax.lax.dot_general(wm, b2_ref[...], (((1,), (0,)), ((), ())),
                                preferred_element_type=f32) + ncfb_ref[0, 0]
        h1 = jax.lax.dot_general(mu_ref[...], W1_ref[:, :D],
                                 (((1,), (1,)), ((), ())),
                                 preferred_element_type=f32)
        h2 = jax.lax.dot_general(mi_ref[...], W1_ref[:, D:],
                                 (((1,), (1,)), ((), ())),
                                 preferred_element_type=f32)
        h = jnp.maximum(h1 + h2 + b1_ref[...], 0.0)
        og = jax.lax.dot_general(g, prod_ref[...], (((1,), (1,)), ((), ())),
                                 preferred_element_type=f32)
        om = jax.lax.dot_general(m, h, (((1,), (1,)), ((), ())),
                                 preferred_element_type=f32)
        out_ref[...] = og + om + c

    full = lambda a: pl.BlockSpec(a.shape, lambda i: (0,) * a.ndim)
    return pl.pallas_call(
        body,
        grid=(grid,),
        in_specs=[
            pl.BlockSpec((R, D), lambda i: (i, 0)),
            pl.BlockSpec((R, D), lambda i: (i, 0)),
            pl.BlockSpec((R, D), lambda i: (i, 0)),
            full(gmf_Wp), full(W1), full(b1), full(W2), full(b2),
            full(ncf_W), full(ncf_b),
        ],
        out_specs=pl.BlockSpec((1, R), lambda i: (0, i)),
        out_shape=jax.ShapeDtypeStruct((1, B), jnp.float32),
    )(prod, mu, mi, gmf_Wp, W1, b1, W2, b2, ncf_W, ncf_b)


def kernel(user_id, item_id, gmf_user_table, gmf_item_table, gmf_W,
           mlp_user_table, mlp_item_table, mlp_W1, mlp_b1, mlp_W2, mlp_b2,
           ncf_W, ncf_b):
    uid = user_id.astype(jnp.int32)
    iid = item_id.astype(jnp.int32)
    B = uid.shape[0]
    D = gmf_user_table.shape[1]
    W1 = mlp_W1.reshape(128, 256)
    b1 = mlp_b1.reshape(1, 128)
    b2 = mlp_b2.reshape(64, 1)
    ncfb = ncf_b.reshape(1, 1)
    prod, mu, mi = _sc_gather(uid, iid, gmf_user_table, gmf_item_table,
                              mlp_user_table, mlp_item_table)
    out = _tc_tail(prod, mu, mi, gmf_W, W1, b1, mlp_W2, b2, ncf_W, ncfb)
    return out.reshape(B, 1)


# speedup vs baseline: 1.0345x; 1.0345x over previous
import functools

import jax
import jax.numpy as jnp
from jax import lax
from jax.experimental import pallas as pl
from jax.experimental.pallas import tpu as pltpu
from jax.experimental.pallas import tpu_sc as plsc

_NC = 2
_NS = 16
_NW = _NC * _NS
_LANES = 16


def _sc_gather(uid, iid, gmf_u, gmf_i, mlp_u, mlp_i):
    B = uid.shape[0]
    D = gmf_u.shape[1]
    bpw = B // _NW
    C = 64
    nchunks = bpw // C
    nbuf = 2

    mesh = plsc.VectorSubcoreMesh(core_axis_name="c", subcore_axis_name="s")
    fdim = jax.ShapeDtypeStruct((B, D), jnp.float32)
    cbuf = pltpu.VMEM((C, D), jnp.float32)
    @functools.partial(
        pl.kernel,
        out_type=[fdim, fdim, fdim],
        mesh=mesh,
        scratch_types=[
            pltpu.VMEM((bpw,), jnp.int32),
            pltpu.VMEM((bpw,), jnp.int32),
            [[cbuf] * 4] * nbuf,
            [pltpu.SemaphoreType.DMA] * nbuf,
            [pltpu.SemaphoreType.DMA] * nbuf,
            [pltpu.SemaphoreType.DMA] * nbuf,
        ],
    )
    def k(uid_hbm, iid_hbm, gu_hbm, gi_hbm, mu_hbm, mi_hbm,
          prod_out, mu_out, mi_out,
          uidx, iidx, bufs, gsem, msem, wsem):
        wid = lax.axis_index("s") * _NC + lax.axis_index("c")
        base = wid * bpw
        pltpu.sync_copy(uid_hbm.at[pl.ds(base, bpw)], uidx)
        pltpu.sync_copy(iid_hbm.at[pl.ds(base, bpw)], iidx)

        def gathers(ci, s):
            off = ci * C
            cu = uidx.at[pl.ds(off, C)]
            cv = iidx.at[pl.ds(off, C)]
            bgu, bgi, bmu, bmi = bufs[s]
            return [pltpu.make_async_copy(gu_hbm.at[cu], bgu, gsem[s]),
                    pltpu.make_async_copy(gi_hbm.at[cv], bgi, gsem[s]),
                    pltpu.make_async_copy(mu_hbm.at[cu], bmu, msem[s]),
                    pltpu.make_async_copy(mi_hbm.at[cv], bmi, msem[s])]

        def writes(ci, s):
            off = base + ci * C
            bgu, _, bmu, bmi = bufs[s]
            return [pltpu.make_async_copy(bgu, prod_out.at[pl.ds(off, C)], wsem[s]),
                    pltpu.make_async_copy(bmu, mu_out.at[pl.ds(off, C)], wsem[s]),
                    pltpu.make_async_copy(bmi, mi_out.at[pl.ds(off, C)], wsem[s])]

        def process(ci, s):
            cg = gathers(ci, s)
            ws = writes(ci, s)
            cg[2].wait()
            cg[3].wait()
            ws[1].start()
            ws[2].start()
            cg[0].wait()
            cg[1].wait()
            bgu, bgi, _, _ = bufs[s]

            def row(j, carry2):
                for t in range(D // _LANES):
                    sl = pl.ds(t * _LANES, _LANES)
                    bgu[j, sl] = bgu[j, sl] * bgi[j, sl]
                return carry2

            lax.fori_loop(0, C, row, 0)
            ws[0].start()

        for ci in range(nchunks):
            s = ci % nbuf
            if ci >= nbuf:
                for c in writes(ci - nbuf, s):
                    c.wait()
            for c in gathers(ci, s):
                c.start()
            if ci > 0:
                process(ci - 1, (ci - 1) % nbuf)
        process(nchunks - 1, (nchunks - 1) % nbuf)
        for ci in range(nchunks - nbuf, nchunks):
            for c in writes(ci, ci % nbuf):
                c.wait()

    return k(uid, iid, gmf_u, gmf_i, mlp_u, mlp_i)


def _tc_tail(prod, mu, mi, gmf_Wp, W1, b1, W2, b2, ncf_W, ncf_b):
    B, D = mu.shape
    R = 4096
    grid = B // R

    def body(prod_ref, mu_ref, mi_ref, gmfW_ref, W1_ref, b1_ref, W2_ref,
             b2_ref, ncfW_ref, ncfb_ref, out_ref):
        f32 = jnp.float32
        wg = ncfW_ref[:, :64]
        wm = ncfW_ref[:, 64:]
        g = jax.lax.dot_general(wg, gmfW_ref[...], (((1,), (0,)), ((), ())),
                                preferred_element_type=f32)
        m = jax.lax.dot_general(wm, W2_ref[...], (((1,), (0,)), ((), ())),
                                preferred_element_type=f32)
        c = jax.lax.dot_general(wm, b2_ref[...], (((1,), (0,)), ((), ())),
                                preferred_element_type=f32) + ncfb_ref[0, 0]
        h1 = jax.lax.dot_general(mu_ref[...], W1_ref[:, :D],
                                 (((1,), (1,)), ((), ())),
                                 preferred_element_type=f32)
        h2 = jax.lax.dot_general(mi_ref[...], W1_ref[:, D:],
                                 (((1,), (1,)), ((), ())),
                                 preferred_element_type=f32)
        h = jnp.maximum(h1 + h2 + b1_ref[...], 0.0)
        og = jax.lax.dot_general(g, prod_ref[...], (((1,), (1,)), ((), ())),
                                 preferred_element_type=f32)
        om = jax.lax.dot_general(m, h, (((1,), (1,)), ((), ())),
                                 preferred_element_type=f32)
        out_ref[...] = og + om + c

    full = lambda a: pl.BlockSpec(a.shape, lambda i: (0,) * a.ndim)
    return pl.pallas_call(
        body,
        grid=(grid,),
        in_specs=[
            pl.BlockSpec((R, D), lambda i: (i, 0)),
            pl.BlockSpec((R, D), lambda i: (i, 0)),
            pl.BlockSpec((R, D), lambda i: (i, 0)),
            full(gmf_Wp), full(W1), full(b1), full(W2), full(b2),
            full(ncf_W), full(ncf_b),
        ],
        out_specs=pl.BlockSpec((1, R), lambda i: (0, i)),
        out_shape=jax.ShapeDtypeStruct((1, B), jnp.float32),
    )(prod, mu, mi, gmf_Wp, W1, b1, W2, b2, ncf_W, ncf_b)


def kernel(user_id, item_id, gmf_user_table, gmf_item_table, gmf_W,
           mlp_user_table, mlp_item_table, mlp_W1, mlp_b1, mlp_W2, mlp_b2,
           ncf_W, ncf_b):
    uid = user_id.astype(jnp.int32)
    iid = item_id.astype(jnp.int32)
    B = uid.shape[0]
    D = gmf_user_table.shape[1]
    W1 = mlp_W1.reshape(128, 256)
    b1 = mlp_b1.reshape(1, 128)
    b2 = mlp_b2.reshape(64, 1)
    ncfb = ncf_b.reshape(1, 1)
    prod, mu, mi = _sc_gather(uid, iid, gmf_user_table, gmf_item_table,
                              mlp_user_table, mlp_item_table)
    out = _tc_tail(prod, mu, mi, gmf_W, W1, b1, mlp_W2, b2, ncf_W, ncfb)
    return out.reshape(B, 1)
